# baseline (speedup 1.0000x reference)
import functools

import jax
import jax.numpy as jnp
from jax import lax
from jax.experimental import pallas as pl
from jax.experimental.pallas import tpu as pltpu
from jax.experimental.pallas import tpu_sc as plsc

_D = 64
_T = 200
_BB = 4096
_BLK = 128
_BLK_PER_T = _BB // _BLK
_N_BLOCKS = _T * _BLK_PER_T

_info = plsc.get_sparse_core_info()
_NC = _info.num_cores
_NS = _info.num_subcores
_NW = _NC * _NS
_BLK_PER_W = _N_BLOCKS // _NW
_IDX_PER_W = _BLK_PER_W * _BLK

_mesh = plsc.VectorSubcoreMesh(core_axis_name="c", subcore_axis_name="s")


@functools.partial(
    pl.kernel,
    mesh=_mesh,
    out_type=jax.ShapeDtypeStruct((_T, _D, _BB), jnp.float32),
    scratch_types=[
        pltpu.VMEM((_IDX_PER_W,), jnp.int32),
        pltpu.VMEM((_BLK,), jnp.int32),
        pltpu.VMEM((_BLK,), jnp.int32),
        pltpu.VMEM((_BLK,), jnp.int32),
        pltpu.VMEM((_BLK,), jnp.int32),
        pltpu.VMEM((_BLK, 2 * _D), jnp.float32),
        pltpu.VMEM((_BLK, 2 * _D), jnp.float32),
        pltpu.VMEM((_D, _BLK), jnp.float32),
        pltpu.VMEM((_D, _BLK), jnp.float32),
        pltpu.SemaphoreType.DMA,
        pltpu.SemaphoreType.DMA,
        pltpu.SemaphoreType.DMA,
        pltpu.SemaphoreType.DMA,
    ],
    compiler_params=pltpu.CompilerParams(needs_layout_passes=False),
)
def _embed_t_major(idx_hbm, table_hbm, out_hbm,
                   idx_v, gidx_a, gidx_b, colb_a, colb_b,
                   gbuf_a, gbuf_b, tbuf_a, tbuf_b,
                   gsem_a, gsem_b, ssem_a, ssem_b):
    wid = lax.axis_index("s") * _NC + lax.axis_index("c")
    blk0 = wid * _BLK_PER_W

    pltpu.sync_copy(idx_hbm.at[pl.ds(wid * _IDX_PER_W, _IDX_PER_W)], idx_v)

    lanes = lax.iota(jnp.int32, 16)

    def prep_and_gather(i_local, gidx, colb, gbuf, gsem):
        off = i_local * _BLK
        for k in range(_BLK // 16):
            v = idx_v[pl.ds(off + k * 16, 16)]
            gidx[pl.ds(k * 16, 16)] = lax.shift_right_logical(v, 1)
            colb[pl.ds(k * 16, 16)] = (v & 1) * _D
        pltpu.async_copy(table_hbm.at[gidx], gbuf, gsem)

    def wait_gather(gidx, gbuf, gsem):
        pltpu.make_async_copy(table_hbm.at[gidx], gbuf, gsem).wait()

    def transpose(gbuf, colb, tbuf):
        for k in range(_BLK // 16):
            rows = lanes + (k * 16)
            cb = colb[pl.ds(k * 16, 16)]

            @plsc.parallel_loop(0, _D, 1, unroll=8)
            def _(d):
                x = plsc.load_gather(gbuf, [rows, cb + d])
                tbuf[d, pl.ds(k * 16, 16)] = x

    def start_store(i_local, tbuf, ssem):
        gid = blk0 + i_local
        t = gid // _BLK_PER_T
        b0 = (gid % _BLK_PER_T) * _BLK
        pltpu.async_copy(tbuf, out_hbm.at[t, :, pl.ds(b0, _BLK)], ssem)

    def wait_store(tbuf, ssem):
        pltpu.make_async_copy(
            tbuf, out_hbm.at[0, :, pl.ds(0, _BLK)], ssem
        ).wait()

    prep_and_gather(0, gidx_a, colb_a, gbuf_a, gsem_a)

    def body(g, _):
        prep_and_gather(2 * g + 1, gidx_b, colb_b, gbuf_b, gsem_b)

        wait_gather(gidx_a, gbuf_a, gsem_a)

        @pl.when(g > 0)
        def _():
            wait_store(tbuf_a, ssem_a)

        transpose(gbuf_a, colb_a, tbuf_a)
        start_store(2 * g, tbuf_a, ssem_a)

        @pl.when(g < _BLK_PER_W // 2 - 1)
        def _():
            prep_and_gather(2 * g + 2, gidx_a, colb_a, gbuf_a, gsem_a)

        wait_gather(gidx_b, gbuf_b, gsem_b)

        @pl.when(g > 0)
        def _():
            wait_store(tbuf_b, ssem_b)

        transpose(gbuf_b, colb_b, tbuf_b)
        start_store(2 * g + 1, tbuf_b, ssem_b)
        return 0

    lax.fori_loop(0, _BLK_PER_W // 2, body, 0)

    wait_store(tbuf_a, ssem_a)
    wait_store(tbuf_b, ssem_b)


def kernel(kmer_indices, embedding_weight):
    idx_t = kmer_indices.T.reshape(-1).astype(jnp.int32)
    table_m = embedding_weight.reshape(-1, 2 * _D)
    outp = _embed_t_major(idx_t, table_m)
    return jnp.transpose(outp, (2, 0, 1))


# speedup vs baseline: 2.2822x; 2.2822x over previous
import functools

import jax
import jax.numpy as jnp
from jax import lax
from jax.experimental import pallas as pl
from jax.experimental.pallas import tpu as pltpu
from jax.experimental.pallas import tpu_sc as plsc

_D = 64
_T = 200
_BB = 4096
_BLK = 128
_BLK_PER_T = _BB // _BLK
_N_BLOCKS = _T * _BLK_PER_T

_info = plsc.get_sparse_core_info()
_NC = _info.num_cores
_NS = _info.num_subcores
_NW = _NC * _NS
_BLK_PER_W = _N_BLOCKS // _NW
_IDX_PER_W = _BLK_PER_W * _BLK

_mesh = plsc.VectorSubcoreMesh(core_axis_name="c", subcore_axis_name="s")


@functools.partial(
    pl.kernel,
    mesh=_mesh,
    out_type=jax.ShapeDtypeStruct((_T, _D, _BB), jnp.float32),
    scratch_types=[
        pltpu.VMEM((_IDX_PER_W,), jnp.int32),
        pltpu.VMEM((_BLK,), jnp.int32),
        pltpu.VMEM((_BLK,), jnp.int32),
        pltpu.VMEM((_BLK, 2 * _D), jnp.float32),
        pltpu.VMEM((_BLK, 2 * _D), jnp.float32),
        pltpu.VMEM((_D, _BLK), jnp.float32),
        pltpu.VMEM((_D, _BLK), jnp.float32),
        pltpu.SemaphoreType.DMA,
        pltpu.SemaphoreType.DMA,
        pltpu.SemaphoreType.DMA,
        pltpu.SemaphoreType.DMA,
    ],
    compiler_params=pltpu.CompilerParams(
        needs_layout_passes=False, disable_bounds_checks=True
    ),
)
def _embed_t_major(idx_hbm, table_hbm, out_hbm,
                   idx_v, gidx_a, gidx_b,
                   gbuf_a, gbuf_b, tbuf_a, tbuf_b,
                   gsem_a, gsem_b, ssem_a, ssem_b):
    wid = lax.axis_index("s") * _NC + lax.axis_index("c")
    blk0 = wid * _BLK_PER_W

    pltpu.sync_copy(idx_hbm.at[pl.ds(wid * _IDX_PER_W, _IDX_PER_W)], idx_v)

    lanes = lax.iota(jnp.int32, 16)

    def prep_and_gather(i_local, gidx, gbuf, gsem):
        off = i_local * _BLK
        for k in range(_BLK // 16):
            v = idx_v[pl.ds(off + k * 16, 16)]
            gidx[pl.ds(k * 16, 16)] = lax.shift_right_logical(v, 1)
        pltpu.async_copy(table_hbm.at[gidx], gbuf, gsem)

    def wait_gather(gidx, gbuf, gsem):
        pltpu.make_async_copy(table_hbm.at[gidx], gbuf, gsem).wait()

    def transpose(i_local, gbuf, tbuf):
        off = i_local * _BLK
        for k in range(_BLK // 16):
            v = idx_v[pl.ds(off + k * 16, 16)]
            cb = (v & 1) * _D
            rows = lanes + (k * 16)

            @plsc.parallel_loop(0, _D, 1, unroll=16)
            def _(d):
                rot = (lanes + d) & (_D - 1)
                x = plsc.load_gather(gbuf, [rows, cb + rot])
                plsc.store_scatter(tbuf, [rot, rows], x)

    def start_store(i_local, tbuf, ssem):
        gid = blk0 + i_local
        t = gid // _BLK_PER_T
        b0 = (gid % _BLK_PER_T) * _BLK
        pltpu.async_copy(tbuf, out_hbm.at[t, :, pl.ds(b0, _BLK)], ssem)

    def wait_store(tbuf, ssem):
        pltpu.make_async_copy(
            tbuf, out_hbm.at[0, :, pl.ds(0, _BLK)], ssem
        ).wait()

    prep_and_gather(0, gidx_a, gbuf_a, gsem_a)

    def body(g, _):
        prep_and_gather(2 * g + 1, gidx_b, gbuf_b, gsem_b)

        wait_gather(gidx_a, gbuf_a, gsem_a)

        @pl.when(g > 0)
        def _():
            wait_store(tbuf_a, ssem_a)

        transpose(2 * g, gbuf_a, tbuf_a)
        start_store(2 * g, tbuf_a, ssem_a)

        @pl.when(g < _BLK_PER_W // 2 - 1)
        def _():
            prep_and_gather(2 * g + 2, gidx_a, gbuf_a, gsem_a)

        wait_gather(gidx_b, gbuf_b, gsem_b)

        @pl.when(g > 0)
        def _():
            wait_store(tbuf_b, ssem_b)

        transpose(2 * g + 1, gbuf_b, tbuf_b)
        start_store(2 * g + 1, tbuf_b, ssem_b)
        return 0

    lax.fori_loop(0, _BLK_PER_W // 2, body, 0)

    wait_store(tbuf_a, ssem_a)
    wait_store(tbuf_b, ssem_b)


def kernel(kmer_indices, embedding_weight):
    idx_t = kmer_indices.T.reshape(-1).astype(jnp.int32)
    table_m = embedding_weight.reshape(-1, 2 * _D)
    outp = _embed_t_major(idx_t, table_m)
    return jnp.transpose(outp, (2, 0, 1))


# speedup vs baseline: 2.3984x; 1.0509x over previous
import functools

import jax
import jax.numpy as jnp
from jax import lax
from jax.experimental import pallas as pl
from jax.experimental.pallas import tpu as pltpu
from jax.experimental.pallas import tpu_sc as plsc

_D = 64
_T = 200
_BB = 4096
_BLK = 128
_BLK_PER_T = _BB // _BLK
_N_BLOCKS = _T * _BLK_PER_T
_NSLOT = 4

_info = plsc.get_sparse_core_info()
_NC = _info.num_cores
_NS = _info.num_subcores
_NW = _NC * _NS
_BLK_PER_W = _N_BLOCKS // _NW
_IDX_PER_W = _BLK_PER_W * _BLK

_mesh = plsc.VectorSubcoreMesh(core_axis_name="c", subcore_axis_name="s")


@functools.partial(
    pl.kernel,
    mesh=_mesh,
    out_type=jax.ShapeDtypeStruct((_T, _D, _BB), jnp.float32),
    scratch_types=[
        pltpu.VMEM((_IDX_PER_W,), jnp.int32),
        [pltpu.VMEM((_BLK,), jnp.int32)] * _NSLOT,
        [pltpu.VMEM((_BLK, 2 * _D), jnp.float32)] * _NSLOT,
        [pltpu.VMEM((_D, _BLK), jnp.float32)] * _NSLOT,
        [pltpu.SemaphoreType.DMA] * _NSLOT,
        [pltpu.SemaphoreType.DMA] * _NSLOT,
    ],
    compiler_params=pltpu.CompilerParams(
        needs_layout_passes=False, disable_bounds_checks=True
    ),
)
def _embed_t_major(idx_hbm, table_hbm, out_hbm,
                   idx_v, gidxs, gbufs, tbufs, gsems, ssems):
    wid = lax.axis_index("s") * _NC + lax.axis_index("c")
    blk0 = wid * _BLK_PER_W

    pltpu.sync_copy(idx_hbm.at[pl.ds(wid * _IDX_PER_W, _IDX_PER_W)], idx_v)

    lanes = lax.iota(jnp.int32, 16)

    def start_gather(i_local, s):
        off = i_local * _BLK
        for k in range(_BLK // 16):
            v = idx_v[pl.ds(off + k * 16, 16)]
            gidxs[s][pl.ds(k * 16, 16)] = lax.shift_right_logical(v, 1)
        pltpu.async_copy(table_hbm.at[gidxs[s]], gbufs[s], gsems[s])

    def wait_gather(i_local, s):
        pltpu.make_async_copy(table_hbm.at[gidxs[s]], gbufs[s], gsems[s]).wait()

    def transpose(i_local, s):
        gbuf, tbuf = gbufs[s], tbufs[s]
        off = i_local * _BLK
        for k in range(_BLK // 16):
            v = idx_v[pl.ds(off + k * 16, 16)]
            cb = (v & 1) * _D
            rows = lanes + (k * 16)

            @plsc.parallel_loop(0, _D, 1, unroll=16)
            def _(d):
                rot = (lanes + d) & (_D - 1)
                x = plsc.load_gather(gbuf, [rows, cb + rot])
                plsc.store_scatter(tbuf, [rot, rows], x)

    def start_store(i_local, s):
        gid = blk0 + i_local
        t = gid // _BLK_PER_T
        b0 = (gid % _BLK_PER_T) * _BLK
        pltpu.async_copy(tbufs[s], out_hbm.at[t, :, pl.ds(b0, _BLK)], ssems[s])

    def wait_store(s):
        pltpu.make_async_copy(
            tbufs[s], out_hbm.at[0, :, pl.ds(0, _BLK)], ssems[s]
        ).wait()

    for s in range(_NSLOT - 1):
        start_gather(s, s)

    def body(g, _):
        i0 = g * _NSLOT
        start_gather(i0 + _NSLOT - 1, _NSLOT - 1)
        for s in range(_NSLOT):
            wait_gather(i0 + s, s)

            @pl.when(g > 0)
            def _():
                wait_store(s)

            transpose(i0 + s, s)
            start_store(i0 + s, s)

            if s < _NSLOT - 1:
                @pl.when(g < _BLK_PER_W // _NSLOT - 1)
                def _():
                    start_gather(i0 + _NSLOT + s, s)

        return 0

    lax.fori_loop(0, _BLK_PER_W // _NSLOT, body, 0)

    for s in range(_NSLOT):
        wait_store(s)


def kernel(kmer_indices, embedding_weight):
    idx_t = kmer_indices.T.reshape(-1).astype(jnp.int32)
    table_m = embedding_weight.reshape(-1, 2 * _D)
    outp = _embed_t_major(idx_t, table_m)
    return jnp.transpose(outp, (2, 0, 1))
